# baseline (speedup 1.0000x reference)
import jax
import jax.numpy as jnp
from jax.experimental import pallas as pl
from jax.experimental.pallas import tpu as pltpu

_N_E = 1024
_D = 64
_BETA = 0.25


def _rq_body(r_ref, e_ref, et_ref, zh_ref, idx_ref, loss_ref, acc_ref):
    g = pl.program_id(0)
    nb = pl.num_programs(0)
    rfull = r_ref[0].reshape(-1, _D)
    e = e_ref[...]
    et = et_ref[...]
    b2_row = jnp.sum(et * et, axis=0, keepdims=True)
    e2 = e + e

    qs, sums = [], []
    half = rfull.shape[0] // 4
    for h in range(4):
        r = rfull[h * half:(h + 1) * half, :]
        a2_col = jnp.sum(r * r, axis=1, keepdims=True)
        s2 = jax.lax.dot_general(r, e2, (((1,), (1,)), ((), ())),
                                 preferred_element_type=jnp.float32)
        d = (a2_col - s2) + b2_row
        vd = jnp.min(d, axis=1, keepdims=True)
        colf = jax.lax.broadcasted_iota(
            jnp.int32, d.shape, 1).astype(jnp.float32)
        idxf = jnp.min(jnp.where(d == vd, colf, float(_N_E)),
                       axis=1, keepdims=True)
        idx_ref[h * half:(h + 1) * half, :] = idxf.astype(jnp.int32)

        onehot = jnp.where(colf == idxf, 1.0, 0.0)
        q = jax.lax.dot_general(onehot, e, (((1,), (0,)), ((), ())),
                                preferred_element_type=jnp.float32)
        qs.append(q)
        sums.append((
            jnp.sum(jnp.sum(q * q, axis=1, keepdims=True),
                    axis=0, keepdims=True)[0, 0],
            jnp.sum(jnp.sum(q * r, axis=1, keepdims=True),
                    axis=0, keepdims=True)[0, 0],
            jnp.sum(a2_col, axis=0, keepdims=True)[0, 0],
        ))
    q = jnp.concatenate(qs, axis=0)
    zh_ref[...] = (4.0 * q).reshape(zh_ref.shape)
    sum_bb = sums[0][0] + sums[1][0] + sums[2][0] + sums[3][0]
    sum_qz = sums[0][1] + sums[1][1] + sums[2][1] + sums[3][1]
    sum_z2 = sums[0][2] + sums[1][2] + sums[2][2] + sums[3][2]

    @pl.when(g == 0)
    def _init():
        acc_ref[0] = sum_bb
        acc_ref[1] = sum_qz
        acc_ref[2] = sum_z2

    @pl.when(g != 0)
    def _acc():
        acc_ref[0] += sum_bb
        acc_ref[1] += sum_qz
        acc_ref[2] += sum_z2

    @pl.when(g == nb - 1)
    def _fin():
        m = jnp.float32(16 * _D * 32 * 32)
        loss_ref[0] = ((1.0 + _BETA) / 4.0) * (
            30.0 * acc_ref[0] - 20.0 * acc_ref[1] + 4.0 * acc_ref[2]) / m


def _rq_call(rv, e, et, interpret=False):
    B, H, W, C = rv.shape
    return pl.pallas_call(
        _rq_body,
        grid=(B,),
        in_specs=[
            pl.BlockSpec((1, H, W, C), lambda g: (g, 0, 0, 0)),
            pl.BlockSpec((_N_E, _D), lambda g: (0, 0)),
            pl.BlockSpec((_D, _N_E), lambda g: (0, 0)),
        ],
        out_specs=[
            pl.BlockSpec((1, H, W, C), lambda g: (g, 0, 0, 0)),
            pl.BlockSpec((H * W, 1), lambda g: (g, 0)),
            pl.BlockSpec(memory_space=pltpu.SMEM),
        ],
        out_shape=[
            jax.ShapeDtypeStruct((B, H, W, C), jnp.float32),
            jax.ShapeDtypeStruct((B * H * W, 1), jnp.int32),
            jax.ShapeDtypeStruct((1,), jnp.float32),
        ],
        scratch_shapes=[pltpu.SMEM((3,), jnp.float32)],
        interpret=interpret,
    )(rv, e, et)


def kernel(z, embedding_weight):
    z = z.astype(jnp.float32)
    B, C, H, W = z.shape
    rv = jnp.transpose(z, (0, 2, 3, 1))
    et = jnp.transpose(embedding_weight, (1, 0))
    zh, idx, loss = _rq_call(rv, embedding_weight, et)
    z_hat = jnp.transpose(zh, (0, 3, 1, 2))
    idx3 = idx.reshape(B, W, W)
    total_idx = jnp.concatenate([idx3, idx3, idx3, idx3], axis=1)
    return (z_hat, loss.reshape(()), total_idx)
